# baseline (device time: 57729 ns/iter reference)
import jax
import jax.numpy as jnp
from jax import lax
from jax.experimental import pallas as pl
from jax.experimental.pallas import tpu as pltpu

N_DEV = 32
N_GRP = 4
GRP = 8
N_BLK = 4
CHK_PER_BLK = 2


def kernel(x, W1, W2):
    m, _ = x.shape
    k_in = x.shape[1]
    hid = W1.shape[1]
    out_n = W2.shape[1]
    rows = m // GRP
    blk_rows = m // N_BLK

    def body(x_ref, w1_ref, w2_ref, out_ref,
             x_bf, w1_bf, w2_bf, partial_chunks, psum_bf, g96_bf,
             a_buf, b_buf, c_buf,
             a_send, a_recv, b_send, b_recv, c_send, c_recv):
        me = lax.axis_index("i")
        g = me // GRP
        r = lax.rem(me, GRP)

        x_bf[...] = x_ref[...].astype(jnp.bfloat16)
        w1_bf[...] = w1_ref[...].astype(jnp.bfloat16)
        w2_bf[...] = w2_ref[...].astype(jnp.bfloat16)

        for k in range(N_BLK):
            b = lax.rem(r // CHK_PER_BLK + k, N_BLK)
            h = jnp.dot(x_bf[pl.ds(b * blk_rows, blk_rows), :], w1_bf[...],
                        preferred_element_type=jnp.float32)
            h = jnp.maximum(h, 0.0).astype(jnp.bfloat16)
            pb = jnp.dot(h, w2_bf[...], preferred_element_type=jnp.float32)
            partial_chunks[pl.ds(b * CHK_PER_BLK, CHK_PER_BLK)] = (
                pb.astype(jnp.bfloat16).reshape(CHK_PER_BLK, rows, out_n))

            for j in range(CHK_PER_BLK):
                c = b * CHK_PER_BLK + j

                @pl.when(c != r)
                def _():
                    rdma = pltpu.make_async_remote_copy(
                        src_ref=partial_chunks.at[c],
                        dst_ref=a_buf.at[r],
                        send_sem=a_send.at[c],
                        recv_sem=a_recv.at[r],
                        device_id=(g * GRP + c,),
                        device_id_type=pl.DeviceIdType.MESH,
                    )
                    rdma.start()

        a_buf[r] = partial_chunks[r]

        for s in range(GRP):

            @pl.when(s != r)
            def _():
                recv = pltpu.make_async_remote_copy(
                    src_ref=a_buf.at[s],
                    dst_ref=a_buf.at[s],
                    send_sem=c_send.at[s],
                    recv_sem=a_recv.at[s],
                    device_id=(me,),
                    device_id_type=pl.DeviceIdType.MESH,
                )
                recv.wait_recv()

        psum_bf[...] = jnp.sum(
            a_buf[...].astype(jnp.float32), axis=0).astype(jnp.bfloat16)

        for dz in range(1, N_GRP):
            gq = lax.rem(g + dz, N_GRP)
            rdma = pltpu.make_async_remote_copy(
                src_ref=psum_bf,
                dst_ref=b_buf.at[g],
                send_sem=b_send.at[gq],
                recv_sem=b_recv.at[g],
                device_id=(gq * GRP + r,),
                device_id_type=pl.DeviceIdType.MESH,
            )
            rdma.start()

        b_buf[g] = psum_bf[...]

        for s in range(N_GRP):

            @pl.when(s != g)
            def _():
                recv = pltpu.make_async_remote_copy(
                    src_ref=b_buf.at[s],
                    dst_ref=b_buf.at[s],
                    send_sem=b_send.at[s],
                    recv_sem=b_recv.at[s],
                    device_id=(me,),
                    device_id_type=pl.DeviceIdType.MESH,
                )
                recv.wait_recv()

        g96_bf[...] = jnp.sum(
            b_buf[...].astype(jnp.float32), axis=0).astype(jnp.bfloat16)

        for o in range(1, GRP):
            rq = lax.rem(r + o, GRP)
            rdma = pltpu.make_async_remote_copy(
                src_ref=g96_bf,
                dst_ref=c_buf.at[r],
                send_sem=c_send.at[rq],
                recv_sem=c_recv.at[r],
                device_id=(g * GRP + rq,),
                device_id_type=pl.DeviceIdType.MESH,
            )
            rdma.start()

        c_buf[r] = g96_bf[...]

        for s in range(GRP):

            @pl.when(s != r)
            def _():
                recv = pltpu.make_async_remote_copy(
                    src_ref=c_buf.at[s],
                    dst_ref=c_buf.at[s],
                    send_sem=a_send.at[s],
                    recv_sem=c_recv.at[s],
                    device_id=(me,),
                    device_id_type=pl.DeviceIdType.MESH,
                )
                recv.wait_recv()

        out_ref[...] = c_buf[...].astype(jnp.float32).reshape(m, out_n)

        for p in range(GRP):

            @pl.when(p != r)
            def _():
                snd = pltpu.make_async_remote_copy(
                    src_ref=a_buf.at[p],
                    dst_ref=a_buf.at[p],
                    send_sem=a_send.at[p],
                    recv_sem=a_recv.at[p],
                    device_id=(me,),
                    device_id_type=pl.DeviceIdType.MESH,
                )
                snd.wait_send()

        for p in range(N_GRP):

            @pl.when(p != g)
            def _():
                snd = pltpu.make_async_remote_copy(
                    src_ref=psum_bf,
                    dst_ref=b_buf.at[p],
                    send_sem=b_send.at[p],
                    recv_sem=b_recv.at[p],
                    device_id=(me,),
                    device_id_type=pl.DeviceIdType.MESH,
                )
                snd.wait_send()

        for p in range(GRP):

            @pl.when(p != r)
            def _():
                snd = pltpu.make_async_remote_copy(
                    src_ref=g96_bf,
                    dst_ref=c_buf.at[p],
                    send_sem=c_send.at[p],
                    recv_sem=c_recv.at[p],
                    device_id=(me,),
                    device_id_type=pl.DeviceIdType.MESH,
                )
                snd.wait_send()

    return pl.pallas_call(
        body,
        out_shape=jax.ShapeDtypeStruct((m, out_n), jnp.float32),
        in_specs=[
            pl.BlockSpec(memory_space=pltpu.VMEM),
            pl.BlockSpec(memory_space=pltpu.VMEM),
            pl.BlockSpec(memory_space=pltpu.VMEM),
        ],
        out_specs=pl.BlockSpec(memory_space=pltpu.VMEM),
        scratch_shapes=[
            pltpu.VMEM((m, k_in), jnp.bfloat16),
            pltpu.VMEM((k_in, hid), jnp.bfloat16),
            pltpu.VMEM((hid, out_n), jnp.bfloat16),
            pltpu.VMEM((GRP, rows, out_n), jnp.bfloat16),
            pltpu.VMEM((rows, out_n), jnp.bfloat16),
            pltpu.VMEM((rows, out_n), jnp.bfloat16),
            pltpu.VMEM((GRP, rows, out_n), jnp.bfloat16),
            pltpu.VMEM((N_GRP, rows, out_n), jnp.bfloat16),
            pltpu.VMEM((GRP, rows, out_n), jnp.bfloat16),
            pltpu.SemaphoreType.DMA((GRP,)),
            pltpu.SemaphoreType.DMA((GRP,)),
            pltpu.SemaphoreType.DMA((N_GRP,)),
            pltpu.SemaphoreType.DMA((N_GRP,)),
            pltpu.SemaphoreType.DMA((GRP,)),
            pltpu.SemaphoreType.DMA((GRP,)),
        ],
    )(x, W1, W2)


# device time: 53993 ns/iter; 1.0692x vs baseline; 1.0692x over previous
import jax
import jax.numpy as jnp
from jax import lax
from jax.experimental import pallas as pl
from jax.experimental.pallas import tpu as pltpu

N_DEV = 32
N_BLK = 4
DEV_PER_BLK = N_DEV // N_BLK


def kernel(x, W1, W2):
    m, _ = x.shape
    k_in = x.shape[1]
    hid = W1.shape[1]
    out_n = W2.shape[1]
    rows = m // N_DEV
    blk_rows = m // N_BLK

    def body(x_ref, w1_ref, w2_ref, out_ref,
             x_bf, w1_bf, w2_bf, partial_chunks, reduced_bf, rs_buf, ag_buf,
             rs_send, rs_recv, ag_send, ag_recv):
        me = lax.axis_index("i")
        my_blk = me // DEV_PER_BLK

        x_bf[...] = x_ref[...].astype(jnp.bfloat16)
        w1_bf[...] = w1_ref[...].astype(jnp.bfloat16)
        w2_bf[...] = w2_ref[...].astype(jnp.bfloat16)

        for k in range(N_BLK):
            b = lax.rem(my_blk + k, N_BLK)
            h = jnp.dot(x_bf[pl.ds(b * blk_rows, blk_rows), :], w1_bf[...],
                        preferred_element_type=jnp.float32)
            h = jnp.maximum(h, 0.0).astype(jnp.bfloat16)
            pb = jnp.dot(h, w2_bf[...], preferred_element_type=jnp.float32)
            partial_chunks[pl.ds(b * DEV_PER_BLK, DEV_PER_BLK)] = (
                pb.astype(jnp.bfloat16).reshape(DEV_PER_BLK, rows, out_n))

            for j in range(DEV_PER_BLK):
                p = b * DEV_PER_BLK + j
                rdma = pltpu.make_async_remote_copy(
                    src_ref=partial_chunks.at[p],
                    dst_ref=rs_buf.at[me],
                    send_sem=rs_send.at[p],
                    recv_sem=rs_recv.at[me],
                    device_id=(p,),
                    device_id_type=pl.DeviceIdType.MESH,
                )
                rdma.start()

        for s in range(N_DEV):
            recv = pltpu.make_async_remote_copy(
                src_ref=rs_buf.at[s],
                dst_ref=rs_buf.at[s],
                send_sem=ag_send.at[s],
                recv_sem=rs_recv.at[s],
                device_id=(me,),
                device_id_type=pl.DeviceIdType.MESH,
            )
            recv.wait_recv()

        reduced_bf[...] = jnp.sum(
            rs_buf[...].astype(jnp.float32), axis=0).astype(jnp.bfloat16)

        for o in range(N_DEV):
            dest = lax.rem(me + o, N_DEV)
            rdma = pltpu.make_async_remote_copy(
                src_ref=reduced_bf,
                dst_ref=ag_buf.at[me],
                send_sem=ag_send.at[dest],
                recv_sem=ag_recv.at[me],
                device_id=(dest,),
                device_id_type=pl.DeviceIdType.MESH,
            )
            rdma.start()

        for p in range(N_DEV):
            snd = pltpu.make_async_remote_copy(
                src_ref=rs_buf.at[p],
                dst_ref=rs_buf.at[p],
                send_sem=rs_send.at[p],
                recv_sem=rs_recv.at[p],
                device_id=(me,),
                device_id_type=pl.DeviceIdType.MESH,
            )
            snd.wait_send()

        for s in range(N_DEV):
            recv = pltpu.make_async_remote_copy(
                src_ref=ag_buf.at[s],
                dst_ref=ag_buf.at[s],
                send_sem=ag_send.at[s],
                recv_sem=ag_recv.at[s],
                device_id=(me,),
                device_id_type=pl.DeviceIdType.MESH,
            )
            recv.wait_recv()

        out_ref[...] = ag_buf[...].astype(jnp.float32).reshape(m, out_n)

        for p in range(N_DEV):
            snd = pltpu.make_async_remote_copy(
                src_ref=reduced_bf,
                dst_ref=ag_buf.at[p],
                send_sem=ag_send.at[p],
                recv_sem=ag_recv.at[p],
                device_id=(me,),
                device_id_type=pl.DeviceIdType.MESH,
            )
            snd.wait_send()

    return pl.pallas_call(
        body,
        out_shape=jax.ShapeDtypeStruct((m, out_n), jnp.float32),
        in_specs=[
            pl.BlockSpec(memory_space=pltpu.VMEM),
            pl.BlockSpec(memory_space=pltpu.VMEM),
            pl.BlockSpec(memory_space=pltpu.VMEM),
        ],
        out_specs=pl.BlockSpec(memory_space=pltpu.VMEM),
        scratch_shapes=[
            pltpu.VMEM((m, k_in), jnp.bfloat16),
            pltpu.VMEM((k_in, hid), jnp.bfloat16),
            pltpu.VMEM((hid, out_n), jnp.bfloat16),
            pltpu.VMEM((N_DEV, rows, out_n), jnp.bfloat16),
            pltpu.VMEM((rows, out_n), jnp.bfloat16),
            pltpu.VMEM((N_DEV, rows, out_n), jnp.bfloat16),
            pltpu.VMEM((N_DEV, rows, out_n), jnp.bfloat16),
            pltpu.SemaphoreType.DMA((N_DEV,)),
            pltpu.SemaphoreType.DMA((N_DEV,)),
            pltpu.SemaphoreType.DMA((N_DEV,)),
            pltpu.SemaphoreType.DMA((N_DEV,)),
        ],
    )(x, W1, W2)
